# Initial kernel scaffold; baseline (speedup 1.0000x reference)
import jax
import jax.numpy as jnp
from jax.experimental import pallas as pl


def kernel(node_feat, coord_feat, edge_index, edge_feat, We1, be1, We2, be2, Wc1, bc1, Wc2, Wn1, bn1, Wn2, bn2):
    raise NotImplementedError("write your pallas kernel here")


# speedup vs baseline: 3.1982x; 3.1982x over previous
import functools

import jax
import jax.numpy as jnp
from jax import lax
from jax.experimental import pallas as pl
from jax.experimental.pallas import tpu as pltpu
from jax.experimental.pallas import tpu_sc as plsc

NW = 32
CHUNK = 128


def _cdiv(a, b):
    return (a + b - 1) // b


def _build_tables(node_feat, coordp, We1a, be1, We1b, bn):
    n = node_feat.shape[0]
    d = node_feat.shape[1]

    def body(nf_ref, cp_ref, wa_ref, ba_ref, wb_ref, t1_ref, t2_ref):
        nf = nf_ref[...]
        t1_ref[:, :d] = jnp.dot(nf, wa_ref[...],
                                preferred_element_type=jnp.float32) + ba_ref[...]
        t1_ref[:, d:] = cp_ref[...]
        t2_ref[:, :d] = jnp.dot(nf, wb_ref[...],
                                preferred_element_type=jnp.float32)
        t2_ref[:, d:] = cp_ref[...]

    grid = (n // bn,)
    out = pl.pallas_call(
        body,
        grid=grid,
        in_specs=[
            pl.BlockSpec((bn, d), lambda i: (i, 0)),
            pl.BlockSpec((bn, 16), lambda i: (i, 0)),
            pl.BlockSpec((d, d), lambda i: (0, 0)),
            pl.BlockSpec((1, d), lambda i: (0, 0)),
            pl.BlockSpec((d, d), lambda i: (0, 0)),
        ],
        out_specs=[
            pl.BlockSpec((bn, d + 16), lambda i: (i, 0)),
            pl.BlockSpec((bn, d + 16), lambda i: (i, 0)),
        ],
        out_shape=[
            jax.ShapeDtypeStruct((n, d + 16), jnp.float32),
            jax.ShapeDtypeStruct((n, d + 16), jnp.float32),
        ],
    )(node_feat, coordp, We1a, be1, We1b)
    return out


def _gather_edges(t1, t2, src, dst, e, d):
    nchunk = e // CHUNK
    per_w = _cdiv(nchunk, NW)
    mesh = plsc.VectorSubcoreMesh(core_axis_name="c", subcore_axis_name="s")

    @functools.partial(
        pl.kernel,
        out_type=(
            jax.ShapeDtypeStruct((e, d), jnp.float32),
            jax.ShapeDtypeStruct((e, 16), jnp.float32),
        ),
        mesh=mesh,
        scratch_types=[
            pltpu.VMEM((CHUNK,), jnp.int32),
            pltpu.VMEM((CHUNK,), jnp.int32),
            pltpu.VMEM((CHUNK, d + 16), jnp.float32),
            pltpu.VMEM((CHUNK, d + 16), jnp.float32),
            pltpu.VMEM((CHUNK, d), jnp.float32),
            pltpu.VMEM((CHUNK, 16), jnp.float32),
            pltpu.SemaphoreType.DMA,
            pltpu.SemaphoreType.DMA,
        ],
        compiler_params=pltpu.CompilerParams(use_tc_tiling_on_sc=False),
    )
    def gath(t1_hbm, t2_hbm, src_hbm, dst_hbm, gsum_hbm, xd_hbm,
             sidx, didx, r1, r2, go, xo, sem1, sem2):
        w = lax.axis_index("s") * 2 + lax.axis_index("c")

        @pl.loop(0, per_w)
        def _(i):
            c = w + NW * i

            @pl.when(c < nchunk)
            def _():
                base = c * CHUNK
                pltpu.sync_copy(src_hbm.at[pl.ds(base, CHUNK)], sidx)
                pltpu.sync_copy(dst_hbm.at[pl.ds(base, CHUNK)], didx)
                cp1 = pltpu.async_copy(t1_hbm.at[sidx], r1, sem1)
                cp2 = pltpu.async_copy(t2_hbm.at[didx], r2, sem2)
                cp1.wait()
                cp2.wait()

                @pl.loop(0, CHUNK)
                def _(r):
                    row = (pl.ds(r, 1),)
                    for j in range(d // 16):
                        sl = row + (pl.ds(16 * j, 16),)
                        go[sl] = r1[sl] + r2[sl]
                    xsl = row + (pl.ds(d, 16),)
                    xo[row + (pl.ds(0, 16),)] = r1[xsl] - r2[xsl]

                pltpu.sync_copy(go, gsum_hbm.at[pl.ds(base, CHUNK)])
                pltpu.sync_copy(xo, xd_hbm.at[pl.ds(base, CHUNK)])

    return gath(t1, t2, src, dst)


def _edge_mlp(gsum, xd, edge_feat, wr, We1e, We2, be2, Wc1, bc1, wc2r, be, h):
    e = gsum.shape[0]
    ed = edge_feat.shape[1]

    def body(g_ref, xd_ref, ef_ref, wr_ref, w1e_ref, w2_ref, b2_ref,
             wc1_ref, bc1_ref, wc2_ref, mh_ref, mx_ref):
        g = g_ref[...]
        xdv = xd_ref[...]
        radial = jnp.sum(xdv * xdv, axis=1, keepdims=True)
        pre = g + radial * wr_ref[...] + jnp.dot(
            ef_ref[...], w1e_ref[...], preferred_element_type=jnp.float32)
        z = pre * jax.nn.sigmoid(pre)
        m = jnp.dot(z, w2_ref[...], preferred_element_type=jnp.float32) + b2_ref[...]
        m = m * jax.nn.sigmoid(m)
        t = jnp.dot(m, wc1_ref[...], preferred_element_type=jnp.float32) + bc1_ref[...]
        t = t * jax.nn.sigmoid(t)
        coef = jnp.sum(t * wc2_ref[...], axis=1, keepdims=True)
        rnorm = coef / (jnp.sqrt(radial) + 1e-30)
        mh_ref[...] = m
        lane = lax.broadcasted_iota(jnp.int32, (1, 16), 1)
        mx_ref[...] = rnorm * xdv + jnp.where(lane == 3, 1.0, 0.0)

    grid = (e // be,)
    return pl.pallas_call(
        body,
        grid=grid,
        in_specs=[
            pl.BlockSpec((be, h), lambda i: (i, 0)),
            pl.BlockSpec((be, 16), lambda i: (i, 0)),
            pl.BlockSpec((be, ed), lambda i: (i, 0)),
            pl.BlockSpec((1, h), lambda i: (0, 0)),
            pl.BlockSpec((ed, h), lambda i: (0, 0)),
            pl.BlockSpec((h, h), lambda i: (0, 0)),
            pl.BlockSpec((1, h), lambda i: (0, 0)),
            pl.BlockSpec((h, h), lambda i: (0, 0)),
            pl.BlockSpec((1, h), lambda i: (0, 0)),
            pl.BlockSpec((1, h), lambda i: (0, 0)),
        ],
        out_specs=[
            pl.BlockSpec((be, h), lambda i: (i, 0)),
            pl.BlockSpec((be, 16), lambda i: (i, 0)),
        ],
        out_shape=[
            jax.ShapeDtypeStruct((e, h), jnp.float32),
            jax.ShapeDtypeStruct((e, 16), jnp.float32),
        ],
    )(gsum, xd, edge_feat, wr, We1e, We2, be2, Wc1, bc1, wc2r)


def _scatter_edges(mh, mx, dst, zh, zx, n, e, h):
    nchunk = e // CHUNK
    per_w = _cdiv(nchunk, NW)
    rows_per_sub = n // 16
    mesh = plsc.VectorSubcoreMesh(core_axis_name="c", subcore_axis_name="s")

    @functools.partial(
        pl.kernel,
        out_type=(
            jax.ShapeDtypeStruct((2, n, h), jnp.float32),
            jax.ShapeDtypeStruct((2, n, 16), jnp.float32),
        ),
        mesh=mesh,
        scratch_types=[
            pltpu.VMEM((CHUNK,), jnp.int32),
            pltpu.VMEM((CHUNK, h), jnp.float32),
            pltpu.VMEM((CHUNK, 16), jnp.float32),
            pltpu.VMEM_SHARED((n, h), jnp.float32),
            pltpu.VMEM_SHARED((n, 16), jnp.float32),
        ],
        compiler_params=pltpu.CompilerParams(use_tc_tiling_on_sc=False),
    )
    def scat(mh_hbm, mx_hbm, dst_hbm, zh_hbm, zx_hbm, oh_hbm, ox_hbm,
             dv, bh, bx, acch, accx):
        c_id = lax.axis_index("c")
        s_id = lax.axis_index("s")
        w = s_id * 2 + c_id
        rb = s_id * rows_per_sub
        pltpu.sync_copy(zh_hbm.at[pl.ds(rb, rows_per_sub)],
                        acch.at[pl.ds(rb, rows_per_sub)])
        pltpu.sync_copy(zx_hbm.at[pl.ds(rb, rows_per_sub)],
                        accx.at[pl.ds(rb, rows_per_sub)])
        plsc.subcore_barrier()

        @pl.loop(0, per_w)
        def _(i):
            c = w + NW * i

            @pl.when(c < nchunk)
            def _():
                base = c * CHUNK
                pltpu.sync_copy(dst_hbm.at[pl.ds(base, CHUNK)], dv)
                pltpu.sync_copy(mh_hbm.at[pl.ds(base, CHUNK)], bh)
                pltpu.sync_copy(mx_hbm.at[pl.ds(base, CHUNK)], bx)
                pltpu.sync_copy(bh, acch.at[dv], add=True)
                pltpu.sync_copy(bx, accx.at[dv], add=True)

        plsc.subcore_barrier()
        pltpu.sync_copy(acch.at[pl.ds(rb, rows_per_sub)],
                        oh_hbm.at[c_id, pl.ds(rb, rows_per_sub)])
        pltpu.sync_copy(accx.at[pl.ds(rb, rows_per_sub)],
                        ox_hbm.at[c_id, pl.ds(rb, rows_per_sub)])

    return scat(mh, mx, dst, zh, zx)


def _node_mlp(node_feat, coordp, h0, h1, x0, x1, Wn1a, Wn1b, bn1, Wn2, bn2,
              bn, h, o):
    n = node_feat.shape[0]
    d = node_feat.shape[1]

    def body(nf_ref, cp_ref, h0_ref, h1_ref, x0_ref, x1_ref,
             wa_ref, wb_ref, b1_ref, w2_ref, b2_ref, hx_ref, co_ref):
        hn = h0_ref[...] + h1_ref[...]
        sx = x0_ref[...] + x1_ref[...]
        cnt = sx[:, 3:4]
        co_ref[...] = cp_ref[...] + sx / jnp.maximum(cnt, 1.0)
        z = (jnp.dot(nf_ref[...], wa_ref[...], preferred_element_type=jnp.float32)
             + jnp.dot(hn, wb_ref[...], preferred_element_type=jnp.float32)
             + b1_ref[...])
        z = z * jax.nn.sigmoid(z)
        hx_ref[...] = jnp.dot(z, w2_ref[...],
                              preferred_element_type=jnp.float32) + b2_ref[...]

    grid = (n // bn,)
    return pl.pallas_call(
        body,
        grid=grid,
        in_specs=[
            pl.BlockSpec((bn, d), lambda i: (i, 0)),
            pl.BlockSpec((bn, 16), lambda i: (i, 0)),
            pl.BlockSpec((bn, h), lambda i: (i, 0)),
            pl.BlockSpec((bn, h), lambda i: (i, 0)),
            pl.BlockSpec((bn, 16), lambda i: (i, 0)),
            pl.BlockSpec((bn, 16), lambda i: (i, 0)),
            pl.BlockSpec((d, h), lambda i: (0, 0)),
            pl.BlockSpec((h, h), lambda i: (0, 0)),
            pl.BlockSpec((1, h), lambda i: (0, 0)),
            pl.BlockSpec((h, o), lambda i: (0, 0)),
            pl.BlockSpec((1, o), lambda i: (0, 0)),
        ],
        out_specs=[
            pl.BlockSpec((bn, o), lambda i: (i, 0)),
            pl.BlockSpec((bn, 16), lambda i: (i, 0)),
        ],
        out_shape=[
            jax.ShapeDtypeStruct((n, o), jnp.float32),
            jax.ShapeDtypeStruct((n, 16), jnp.float32),
        ],
    )(node_feat, coordp, h0, h1, x0, x1, Wn1a, Wn1b, bn1, Wn2, bn2)


def kernel(node_feat, coord_feat, edge_index, edge_feat,
           We1, be1, We2, be2, Wc1, bc1, Wc2, Wn1, bn1, Wn2, bn2):
    n, d = node_feat.shape
    e = edge_index.shape[1]
    h = We2.shape[0]
    o = Wn2.shape[1]
    ed = edge_feat.shape[1]

    src = edge_index[0]
    dst = edge_index[1]
    coordp = jnp.concatenate(
        [coord_feat, jnp.zeros((n, 16 - coord_feat.shape[1]), jnp.float32)],
        axis=1)

    We1a = We1[:d]
    We1b = We1[d:2 * d]
    wr = We1[2 * d:2 * d + 1]
    We1e = We1[2 * d + 1:]

    t1, t2 = _build_tables(node_feat, coordp, We1a, be1.reshape(1, h), We1b,
                           bn=2000)
    gsum, xd = _gather_edges(t1, t2, src, dst, e, d)
    mh, mx = _edge_mlp(gsum, xd, edge_feat, wr, We1e, We2, be2.reshape(1, h),
                       Wc1, bc1.reshape(1, h), Wc2.reshape(1, h), be=2000, h=h)
    zh = jnp.zeros((n, h), jnp.float32)
    zx = jnp.zeros((n, 16), jnp.float32)
    oh, ox = _scatter_edges(mh, mx, dst, zh, zx, n, e, h)
    hx, co = _node_mlp(node_feat, coordp, oh[0], oh[1], ox[0], ox[1],
                       Wn1[:d], Wn1[d:], bn1.reshape(1, h), Wn2,
                       bn2.reshape(1, o), bn=2000, h=h, o=o)
    return hx, co[:, :coord_feat.shape[1]]


# speedup vs baseline: 3.2483x; 1.0157x over previous
import functools

import jax
import jax.numpy as jnp
from jax import lax
from jax.experimental import pallas as pl
from jax.experimental.pallas import tpu as pltpu
from jax.experimental.pallas import tpu_sc as plsc

NW = 32
CHUNK = 128


def _cdiv(a, b):
    return (a + b - 1) // b


def _build_tables(node_feat, coordp, We1a, be1, We1b, bn):
    n = node_feat.shape[0]
    d = node_feat.shape[1]

    def body(nf_ref, cp_ref, wa_ref, ba_ref, wb_ref, t1_ref, t2_ref):
        nf = nf_ref[...]
        t1_ref[:, :d] = jnp.dot(nf, wa_ref[...],
                                preferred_element_type=jnp.float32) + ba_ref[...]
        t1_ref[:, d:] = cp_ref[...]
        t2_ref[:, :d] = jnp.dot(nf, wb_ref[...],
                                preferred_element_type=jnp.float32)
        t2_ref[:, d:] = cp_ref[...]

    grid = (n // bn,)
    out = pl.pallas_call(
        body,
        grid=grid,
        in_specs=[
            pl.BlockSpec((bn, d), lambda i: (i, 0)),
            pl.BlockSpec((bn, 16), lambda i: (i, 0)),
            pl.BlockSpec((d, d), lambda i: (0, 0)),
            pl.BlockSpec((1, d), lambda i: (0, 0)),
            pl.BlockSpec((d, d), lambda i: (0, 0)),
        ],
        out_specs=[
            pl.BlockSpec((bn, d + 16), lambda i: (i, 0)),
            pl.BlockSpec((bn, d + 16), lambda i: (i, 0)),
        ],
        out_shape=[
            jax.ShapeDtypeStruct((n, d + 16), jnp.float32),
            jax.ShapeDtypeStruct((n, d + 16), jnp.float32),
        ],
    )(node_feat, coordp, We1a, be1, We1b)
    return out


def _gather_edges(t1, t2, src2d, dst2d, e_pad, d):
    nchunk = e_pad // CHUNK
    per_w = nchunk // NW
    w16 = d + 16
    mesh = plsc.VectorSubcoreMesh(core_axis_name="c", subcore_axis_name="s")

    @functools.partial(
        pl.kernel,
        out_type=jax.ShapeDtypeStruct((e_pad, w16), jnp.float32),
        mesh=mesh,
        scratch_types=[
            pltpu.VMEM((per_w, CHUNK), jnp.int32),
            pltpu.VMEM((per_w, CHUNK), jnp.int32),
            pltpu.VMEM((2, CHUNK, w16), jnp.float32),
            pltpu.VMEM((2, CHUNK, w16), jnp.float32),
            pltpu.SemaphoreType.DMA,
            pltpu.SemaphoreType.DMA,
            pltpu.SemaphoreType.DMA,
            pltpu.SemaphoreType.DMA,
        ],
        compiler_params=pltpu.CompilerParams(use_tc_tiling_on_sc=False),
    )
    def gath(t1_hbm, t2_hbm, src_hbm, dst_hbm, gx_hbm,
             sidx, didx, r1, r2, s1a, s1b, s2a, s2b):
        w = lax.axis_index("s") * 2 + lax.axis_index("c")
        cbase = w * per_w
        pltpu.sync_copy(src_hbm.at[pl.ds(cbase, per_w)], sidx)
        pltpu.sync_copy(dst_hbm.at[pl.ds(cbase, per_w)], didx)
        sems1 = (s1a, s1b)
        sems2 = (s2a, s2b)

        @pl.loop(0, per_w)
        def _(c):
            cp1 = pltpu.async_copy(t1_hbm.at[sidx.at[c]], r1.at[0], sems1[0])
            cp2 = pltpu.async_copy(t2_hbm.at[didx.at[c]], r2.at[0], sems2[0])
            cp1.wait()
            cp2.wait()

            @pl.loop(0, CHUNK)
            def _(r):
                row = (0, pl.ds(r, 1))
                for j in range(d // 16):
                    sl = row + (pl.ds(16 * j, 16),)
                    r1[sl] = r1[sl] + r2[sl]
                xsl = row + (pl.ds(d, 16),)
                r1[xsl] = r1[xsl] - r2[xsl]

            pltpu.sync_copy(
                r1.at[0], gx_hbm.at[pl.ds((cbase + c) * CHUNK, CHUNK)])

    return gath(t1, t2, src2d, dst2d)


def _edge_mlp(gx, edge_feat, wr, We1e, We2, be2, Wc1, bc1, wc2r, be, h):
    e = edge_feat.shape[0]
    ed = edge_feat.shape[1]

    def body(gx_ref, ef_ref, wr_ref, w1e_ref, w2_ref, b2_ref,
             wc1_ref, bc1_ref, wc2_ref, mh_ref, mx_ref):
        g = gx_ref[:, :h]
        xdv = gx_ref[:, h:]
        radial = jnp.sum(xdv * xdv, axis=1, keepdims=True)
        pre = g + radial * wr_ref[...] + jnp.dot(
            ef_ref[...], w1e_ref[...], preferred_element_type=jnp.float32)
        z = pre * jax.nn.sigmoid(pre)
        m = jnp.dot(z, w2_ref[...], preferred_element_type=jnp.float32) + b2_ref[...]
        m = m * jax.nn.sigmoid(m)
        t = jnp.dot(m, wc1_ref[...], preferred_element_type=jnp.float32) + bc1_ref[...]
        t = t * jax.nn.sigmoid(t)
        coef = jnp.sum(t * wc2_ref[...], axis=1, keepdims=True)
        rnorm = coef / (jnp.sqrt(radial) + 1e-30)
        mh_ref[...] = m
        lane = lax.broadcasted_iota(jnp.int32, (1, 16), 1)
        mx_ref[...] = rnorm * xdv + jnp.where(lane == 3, 1.0, 0.0)

    grid = (e // be,)
    return pl.pallas_call(
        body,
        grid=grid,
        in_specs=[
            pl.BlockSpec((be, h + 16), lambda i: (i, 0)),
            pl.BlockSpec((be, ed), lambda i: (i, 0)),
            pl.BlockSpec((1, h), lambda i: (0, 0)),
            pl.BlockSpec((ed, h), lambda i: (0, 0)),
            pl.BlockSpec((h, h), lambda i: (0, 0)),
            pl.BlockSpec((1, h), lambda i: (0, 0)),
            pl.BlockSpec((h, h), lambda i: (0, 0)),
            pl.BlockSpec((1, h), lambda i: (0, 0)),
            pl.BlockSpec((1, h), lambda i: (0, 0)),
        ],
        out_specs=[
            pl.BlockSpec((be, h), lambda i: (i, 0)),
            pl.BlockSpec((be, 16), lambda i: (i, 0)),
        ],
        out_shape=[
            jax.ShapeDtypeStruct((e, h), jnp.float32),
            jax.ShapeDtypeStruct((e, 16), jnp.float32),
        ],
    )(gx, edge_feat, wr, We1e, We2, be2, Wc1, bc1, wc2r)


def _scatter_edges(mh, mx, dst2d, zh, zx, n, e, e_pad, h):
    nchunk = e // CHUNK
    nchunk_pad = e_pad // CHUNK
    per_w = nchunk_pad // NW
    rows_per_sub = n // 16
    mesh = plsc.VectorSubcoreMesh(core_axis_name="c", subcore_axis_name="s")

    @functools.partial(
        pl.kernel,
        out_type=(
            jax.ShapeDtypeStruct((2, n, h), jnp.float32),
            jax.ShapeDtypeStruct((2, n, 16), jnp.float32),
        ),
        mesh=mesh,
        scratch_types=[
            pltpu.VMEM((2, CHUNK), jnp.int32),
            pltpu.VMEM((2, CHUNK, h), jnp.float32),
            pltpu.VMEM((2, CHUNK, 16), jnp.float32),
            pltpu.VMEM_SHARED((n, h), jnp.float32),
            pltpu.VMEM_SHARED((n, 16), jnp.float32),
            pltpu.SemaphoreType.DMA,
            pltpu.SemaphoreType.DMA,
            pltpu.SemaphoreType.DMA,
            pltpu.SemaphoreType.DMA,
            pltpu.SemaphoreType.DMA,
            pltpu.SemaphoreType.DMA,
        ],
        compiler_params=pltpu.CompilerParams(use_tc_tiling_on_sc=False),
    )
    def scat(mh_hbm, mx_hbm, dst_hbm, zh_hbm, zx_hbm, oh_hbm, ox_hbm,
             dv, bh, bx, acch, accx, sha, shb, sxa, sxb, sdva, sdvb):
        c_id = lax.axis_index("c")
        s_id = lax.axis_index("s")
        w = s_id * 2 + c_id
        cbase = w * per_w
        rb = s_id * rows_per_sub
        pltpu.sync_copy(zh_hbm.at[pl.ds(rb, rows_per_sub)],
                        acch.at[pl.ds(rb, rows_per_sub)])
        pltpu.sync_copy(zx_hbm.at[pl.ds(rb, rows_per_sub)],
                        accx.at[pl.ds(rb, rows_per_sub)])
        plsc.subcore_barrier()

        semsh = (sha, shb)
        semsx = (sxa, sxb)
        semsd = (sdva, sdvb)

        @pl.loop(0, per_w)
        def _(i):
            @pl.when(cbase + i < nchunk)
            def _():
                base = (cbase + i) * CHUNK
                pltpu.sync_copy(dst_hbm.at[cbase + i], dv.at[0])
                pltpu.sync_copy(mh_hbm.at[pl.ds(base, CHUNK)], bh.at[0])
                pltpu.sync_copy(mx_hbm.at[pl.ds(base, CHUNK)], bx.at[0])
                pltpu.sync_copy(bh.at[0], acch.at[dv.at[0]], add=True)
                pltpu.sync_copy(bx.at[0], accx.at[dv.at[0]], add=True)

        plsc.subcore_barrier()
        pltpu.sync_copy(acch.at[pl.ds(rb, rows_per_sub)],
                        oh_hbm.at[c_id, pl.ds(rb, rows_per_sub)])
        pltpu.sync_copy(accx.at[pl.ds(rb, rows_per_sub)],
                        ox_hbm.at[c_id, pl.ds(rb, rows_per_sub)])

    return scat(mh, mx, dst2d, zh, zx)


def _node_mlp(node_feat, coordp, h0, h1, x0, x1, Wn1a, Wn1b, bn1, Wn2, bn2,
              bn, h, o):
    n = node_feat.shape[0]
    d = node_feat.shape[1]

    def body(nf_ref, cp_ref, h0_ref, h1_ref, x0_ref, x1_ref,
             wa_ref, wb_ref, b1_ref, w2_ref, b2_ref, hx_ref, co_ref):
        hn = h0_ref[...] + h1_ref[...]
        sx = x0_ref[...] + x1_ref[...]
        cnt = sx[:, 3:4]
        co_ref[...] = cp_ref[...] + sx / jnp.maximum(cnt, 1.0)
        z = (jnp.dot(nf_ref[...], wa_ref[...], preferred_element_type=jnp.float32)
             + jnp.dot(hn, wb_ref[...], preferred_element_type=jnp.float32)
             + b1_ref[...])
        z = z * jax.nn.sigmoid(z)
        hx_ref[...] = jnp.dot(z, w2_ref[...],
                              preferred_element_type=jnp.float32) + b2_ref[...]

    grid = (n // bn,)
    return pl.pallas_call(
        body,
        grid=grid,
        in_specs=[
            pl.BlockSpec((bn, d), lambda i: (i, 0)),
            pl.BlockSpec((bn, 16), lambda i: (i, 0)),
            pl.BlockSpec((bn, h), lambda i: (i, 0)),
            pl.BlockSpec((bn, h), lambda i: (i, 0)),
            pl.BlockSpec((bn, 16), lambda i: (i, 0)),
            pl.BlockSpec((bn, 16), lambda i: (i, 0)),
            pl.BlockSpec((d, h), lambda i: (0, 0)),
            pl.BlockSpec((h, h), lambda i: (0, 0)),
            pl.BlockSpec((1, h), lambda i: (0, 0)),
            pl.BlockSpec((h, o), lambda i: (0, 0)),
            pl.BlockSpec((1, o), lambda i: (0, 0)),
        ],
        out_specs=[
            pl.BlockSpec((bn, o), lambda i: (i, 0)),
            pl.BlockSpec((bn, 16), lambda i: (i, 0)),
        ],
        out_shape=[
            jax.ShapeDtypeStruct((n, o), jnp.float32),
            jax.ShapeDtypeStruct((n, 16), jnp.float32),
        ],
    )(node_feat, coordp, h0, h1, x0, x1, Wn1a, Wn1b, bn1, Wn2, bn2)


def kernel(node_feat, coord_feat, edge_index, edge_feat,
           We1, be1, We2, be2, Wc1, bc1, Wc2, Wn1, bn1, Wn2, bn2):
    n, d = node_feat.shape
    e = edge_index.shape[1]
    h = We2.shape[0]
    o = Wn2.shape[1]

    nchunk = _cdiv(e, CHUNK)
    nchunk_pad = _cdiv(nchunk, NW) * NW
    e_pad = nchunk_pad * CHUNK

    pad = jnp.zeros((2, e_pad - e), jnp.int32)
    eidx = jnp.concatenate([edge_index, pad], axis=1)
    src2d = eidx[0].reshape(nchunk_pad, CHUNK)
    dst2d = eidx[1].reshape(nchunk_pad, CHUNK)

    coordp = jnp.concatenate(
        [coord_feat, jnp.zeros((n, 16 - coord_feat.shape[1]), jnp.float32)],
        axis=1)

    We1a = We1[:d]
    We1b = We1[d:2 * d]
    wr = We1[2 * d:2 * d + 1]
    We1e = We1[2 * d + 1:]

    t1, t2 = _build_tables(node_feat, coordp, We1a, be1.reshape(1, h), We1b,
                           bn=2000)
    gx = _gather_edges(t1, t2, src2d, dst2d, e_pad, d)
    mh, mx = _edge_mlp(gx, edge_feat, wr, We1e, We2, be2.reshape(1, h),
                       Wc1, bc1.reshape(1, h), Wc2.reshape(1, h), be=2000, h=h)
    zh = jnp.zeros((n, h), jnp.float32)
    zx = jnp.zeros((n, 16), jnp.float32)
    oh, ox = _scatter_edges(mh, mx, dst2d, zh, zx, n, e, e_pad, h)
    hx, co = _node_mlp(node_feat, coordp, oh[0], oh[1], ox[0], ox[1],
                       Wn1[:d], Wn1[d:], bn1.reshape(1, h), Wn2,
                       bn2.reshape(1, o), bn=2000, h=h, o=o)
    return hx, co[:, :coord_feat.shape[1]]


# speedup vs baseline: 3.4944x; 1.0758x over previous
import functools

import jax
import jax.numpy as jnp
from jax import lax
from jax.experimental import pallas as pl
from jax.experimental.pallas import tpu as pltpu
from jax.experimental.pallas import tpu_sc as plsc

NW = 32
CHUNK = 128


def _cdiv(a, b):
    return (a + b - 1) // b


def _build_tables(node_feat, coordp, We1a, be1, We1b, bn):
    n = node_feat.shape[0]
    d = node_feat.shape[1]

    def body(nf_ref, cp_ref, wa_ref, ba_ref, wb_ref, t1_ref, t2_ref):
        nf = nf_ref[...]
        t1_ref[:, :d] = jnp.dot(nf, wa_ref[...],
                                preferred_element_type=jnp.float32) + ba_ref[...]
        t1_ref[:, d:] = cp_ref[...]
        t2_ref[:, :d] = jnp.dot(nf, wb_ref[...],
                                preferred_element_type=jnp.float32)
        t2_ref[:, d:] = cp_ref[...]

    grid = (n // bn,)
    out = pl.pallas_call(
        body,
        grid=grid,
        in_specs=[
            pl.BlockSpec((bn, d), lambda i: (i, 0)),
            pl.BlockSpec((bn, 16), lambda i: (i, 0)),
            pl.BlockSpec((d, d), lambda i: (0, 0)),
            pl.BlockSpec((1, d), lambda i: (0, 0)),
            pl.BlockSpec((d, d), lambda i: (0, 0)),
        ],
        out_specs=[
            pl.BlockSpec((bn, d + 16), lambda i: (i, 0)),
            pl.BlockSpec((bn, d + 16), lambda i: (i, 0)),
        ],
        out_shape=[
            jax.ShapeDtypeStruct((n, d + 16), jnp.float32),
            jax.ShapeDtypeStruct((n, d + 16), jnp.float32),
        ],
    )(node_feat, coordp, We1a, be1, We1b)
    return out


def _gather_edges(t1, t2, src2d, dst2d, e_pad, d):
    nchunk = e_pad // CHUNK
    per_w = nchunk // NW
    w16 = d + 16
    mesh = plsc.VectorSubcoreMesh(core_axis_name="c", subcore_axis_name="s")

    @functools.partial(
        pl.kernel,
        out_type=jax.ShapeDtypeStruct((e_pad, w16), jnp.float32),
        mesh=mesh,
        scratch_types=[
            pltpu.VMEM((per_w, CHUNK), jnp.int32),
            pltpu.VMEM((per_w, CHUNK), jnp.int32),
            pltpu.VMEM((2, CHUNK, w16), jnp.float32),
            pltpu.VMEM((2, CHUNK, w16), jnp.float32),
            pltpu.SemaphoreType.DMA,
            pltpu.SemaphoreType.DMA,
            pltpu.SemaphoreType.DMA,
            pltpu.SemaphoreType.DMA,
        ],
        compiler_params=pltpu.CompilerParams(use_tc_tiling_on_sc=False),
    )
    def gath(t1_hbm, t2_hbm, src_hbm, dst_hbm, gx_hbm,
             sidx, didx, r1, r2, s1a, s1b, s2a, s2b):
        w = lax.axis_index("s") * 2 + lax.axis_index("c")
        cbase = w * per_w
        pltpu.sync_copy(src_hbm.at[pl.ds(cbase, per_w)], sidx)
        pltpu.sync_copy(dst_hbm.at[pl.ds(cbase, per_w)], didx)
        sems1 = (s1a, s1b)
        sems2 = (s2a, s2b)

        def issue(c, b):
            pltpu.async_copy(t1_hbm.at[sidx.at[c]], r1.at[b], sems1[b])
            pltpu.async_copy(t2_hbm.at[didx.at[c]], r2.at[b], sems2[b])

        def wait(b):
            pltpu.make_async_copy(t1_hbm.at[sidx.at[0]], r1.at[b],
                                  sems1[b]).wait()
            pltpu.make_async_copy(t2_hbm.at[didx.at[0]], r2.at[b],
                                  sems2[b]).wait()

        issue(0, 0)
        issue(1, 1)

        @pl.loop(0, _cdiv(per_w, 2))
        def _(i):
            for b in range(2):
                c = 2 * i + b

                @pl.when(c < per_w)
                def _():
                    wait(b)

                    @pl.loop(0, CHUNK)
                    def _(r):
                        row = (b, pl.ds(r, 1))
                        for j in range(d // 16):
                            sl = row + (pl.ds(16 * j, 16),)
                            r1[sl] = r1[sl] + r2[sl]
                        xsl = row + (pl.ds(d, 16),)
                        r1[xsl] = r1[xsl] - r2[xsl]

                    pltpu.sync_copy(
                        r1.at[b], gx_hbm.at[pl.ds((cbase + c) * CHUNK, CHUNK)])

                    @pl.when(c + 2 < per_w)
                    def _():
                        issue(c + 2, b)

    return gath(t1, t2, src2d, dst2d)


def _edge_mlp(gx, edge_feat, wr, We1e, We2, be2, Wc1, bc1, wc2r, be, h):
    e = edge_feat.shape[0]
    ed = edge_feat.shape[1]

    def body(gx_ref, ef_ref, wr_ref, w1e_ref, w2_ref, b2_ref,
             wc1_ref, bc1_ref, wc2_ref, mh_ref, mx_ref):
        g = gx_ref[:, :h]
        xdv = gx_ref[:, h:]
        radial = jnp.sum(xdv * xdv, axis=1, keepdims=True)
        pre = g + radial * wr_ref[...] + jnp.dot(
            ef_ref[...], w1e_ref[...], preferred_element_type=jnp.float32)
        z = pre * jax.nn.sigmoid(pre)
        m = jnp.dot(z, w2_ref[...], preferred_element_type=jnp.float32) + b2_ref[...]
        m = m * jax.nn.sigmoid(m)
        t = jnp.dot(m, wc1_ref[...], preferred_element_type=jnp.float32) + bc1_ref[...]
        t = t * jax.nn.sigmoid(t)
        coef = jnp.sum(t * wc2_ref[...], axis=1, keepdims=True)
        rnorm = coef / (jnp.sqrt(radial) + 1e-30)
        mh_ref[...] = m
        lane = lax.broadcasted_iota(jnp.int32, (1, 16), 1)
        mx_ref[...] = rnorm * xdv + jnp.where(lane == 3, 1.0, 0.0)

    grid = (e // be,)
    return pl.pallas_call(
        body,
        grid=grid,
        in_specs=[
            pl.BlockSpec((be, h + 16), lambda i: (i, 0)),
            pl.BlockSpec((be, ed), lambda i: (i, 0)),
            pl.BlockSpec((1, h), lambda i: (0, 0)),
            pl.BlockSpec((ed, h), lambda i: (0, 0)),
            pl.BlockSpec((h, h), lambda i: (0, 0)),
            pl.BlockSpec((1, h), lambda i: (0, 0)),
            pl.BlockSpec((h, h), lambda i: (0, 0)),
            pl.BlockSpec((1, h), lambda i: (0, 0)),
            pl.BlockSpec((1, h), lambda i: (0, 0)),
        ],
        out_specs=[
            pl.BlockSpec((be, h), lambda i: (i, 0)),
            pl.BlockSpec((be, 16), lambda i: (i, 0)),
        ],
        out_shape=[
            jax.ShapeDtypeStruct((e, h), jnp.float32),
            jax.ShapeDtypeStruct((e, 16), jnp.float32),
        ],
    )(gx, edge_feat, wr, We1e, We2, be2, Wc1, bc1, wc2r)


def _scatter_edges(mh, mx, dst2d, zh, zx, n, e, e_pad, h):
    nchunk = e // CHUNK
    nchunk_pad = e_pad // CHUNK
    per_w = nchunk_pad // NW
    rows_per_sub = n // 16
    mesh = plsc.VectorSubcoreMesh(core_axis_name="c", subcore_axis_name="s")

    @functools.partial(
        pl.kernel,
        out_type=(
            jax.ShapeDtypeStruct((2, n, h), jnp.float32),
            jax.ShapeDtypeStruct((2, n, 16), jnp.float32),
        ),
        mesh=mesh,
        scratch_types=[
            pltpu.VMEM((2, CHUNK), jnp.int32),
            pltpu.VMEM((2, CHUNK, h), jnp.float32),
            pltpu.VMEM((2, CHUNK, 16), jnp.float32),
            pltpu.VMEM_SHARED((n, h), jnp.float32),
            pltpu.VMEM_SHARED((n, 16), jnp.float32),
            pltpu.SemaphoreType.DMA,
            pltpu.SemaphoreType.DMA,
            pltpu.SemaphoreType.DMA,
            pltpu.SemaphoreType.DMA,
            pltpu.SemaphoreType.DMA,
            pltpu.SemaphoreType.DMA,
        ],
        compiler_params=pltpu.CompilerParams(use_tc_tiling_on_sc=False),
    )
    def scat(mh_hbm, mx_hbm, dst_hbm, zh_hbm, zx_hbm, oh_hbm, ox_hbm,
             dv, bh, bx, acch, accx, sha, shb, sxa, sxb, sdva, sdvb):
        c_id = lax.axis_index("c")
        s_id = lax.axis_index("s")
        w = s_id * 2 + c_id
        cbase = w * per_w
        rb = s_id * rows_per_sub
        pltpu.sync_copy(zh_hbm.at[pl.ds(rb, rows_per_sub)],
                        acch.at[pl.ds(rb, rows_per_sub)])
        pltpu.sync_copy(zx_hbm.at[pl.ds(rb, rows_per_sub)],
                        accx.at[pl.ds(rb, rows_per_sub)])
        plsc.subcore_barrier()

        semsh = (sha, shb)
        semsx = (sxa, sxb)
        semsd = (sdva, sdvb)

        @pl.loop(0, per_w)
        def _(i):
            @pl.when(cbase + i < nchunk)
            def _():
                base = (cbase + i) * CHUNK
                pltpu.sync_copy(dst_hbm.at[cbase + i], dv.at[0])
                pltpu.sync_copy(mh_hbm.at[pl.ds(base, CHUNK)], bh.at[0])
                pltpu.sync_copy(mx_hbm.at[pl.ds(base, CHUNK)], bx.at[0])
                pltpu.sync_copy(bh.at[0], acch.at[dv.at[0]], add=True)
                pltpu.sync_copy(bx.at[0], accx.at[dv.at[0]], add=True)

        plsc.subcore_barrier()
        pltpu.sync_copy(acch.at[pl.ds(rb, rows_per_sub)],
                        oh_hbm.at[c_id, pl.ds(rb, rows_per_sub)])
        pltpu.sync_copy(accx.at[pl.ds(rb, rows_per_sub)],
                        ox_hbm.at[c_id, pl.ds(rb, rows_per_sub)])

    return scat(mh, mx, dst2d, zh, zx)


def _node_mlp(node_feat, coordp, h0, h1, x0, x1, Wn1a, Wn1b, bn1, Wn2, bn2,
              bn, h, o):
    n = node_feat.shape[0]
    d = node_feat.shape[1]

    def body(nf_ref, cp_ref, h0_ref, h1_ref, x0_ref, x1_ref,
             wa_ref, wb_ref, b1_ref, w2_ref, b2_ref, hx_ref, co_ref):
        hn = h0_ref[...] + h1_ref[...]
        sx = x0_ref[...] + x1_ref[...]
        cnt = sx[:, 3:4]
        co_ref[...] = cp_ref[...] + sx / jnp.maximum(cnt, 1.0)
        z = (jnp.dot(nf_ref[...], wa_ref[...], preferred_element_type=jnp.float32)
             + jnp.dot(hn, wb_ref[...], preferred_element_type=jnp.float32)
             + b1_ref[...])
        z = z * jax.nn.sigmoid(z)
        hx_ref[...] = jnp.dot(z, w2_ref[...],
                              preferred_element_type=jnp.float32) + b2_ref[...]

    grid = (n // bn,)
    return pl.pallas_call(
        body,
        grid=grid,
        in_specs=[
            pl.BlockSpec((bn, d), lambda i: (i, 0)),
            pl.BlockSpec((bn, 16), lambda i: (i, 0)),
            pl.BlockSpec((bn, h), lambda i: (i, 0)),
            pl.BlockSpec((bn, h), lambda i: (i, 0)),
            pl.BlockSpec((bn, 16), lambda i: (i, 0)),
            pl.BlockSpec((bn, 16), lambda i: (i, 0)),
            pl.BlockSpec((d, h), lambda i: (0, 0)),
            pl.BlockSpec((h, h), lambda i: (0, 0)),
            pl.BlockSpec((1, h), lambda i: (0, 0)),
            pl.BlockSpec((h, o), lambda i: (0, 0)),
            pl.BlockSpec((1, o), lambda i: (0, 0)),
        ],
        out_specs=[
            pl.BlockSpec((bn, o), lambda i: (i, 0)),
            pl.BlockSpec((bn, 16), lambda i: (i, 0)),
        ],
        out_shape=[
            jax.ShapeDtypeStruct((n, o), jnp.float32),
            jax.ShapeDtypeStruct((n, 16), jnp.float32),
        ],
    )(node_feat, coordp, h0, h1, x0, x1, Wn1a, Wn1b, bn1, Wn2, bn2)


def kernel(node_feat, coord_feat, edge_index, edge_feat,
           We1, be1, We2, be2, Wc1, bc1, Wc2, Wn1, bn1, Wn2, bn2):
    n, d = node_feat.shape
    e = edge_index.shape[1]
    h = We2.shape[0]
    o = Wn2.shape[1]

    nchunk = _cdiv(e, CHUNK)
    nchunk_pad = _cdiv(nchunk, NW) * NW
    e_pad = nchunk_pad * CHUNK

    pad = jnp.zeros((2, e_pad - e), jnp.int32)
    eidx = jnp.concatenate([edge_index, pad], axis=1)
    src2d = eidx[0].reshape(nchunk_pad, CHUNK)
    dst2d = eidx[1].reshape(nchunk_pad, CHUNK)

    coordp = jnp.concatenate(
        [coord_feat, jnp.zeros((n, 16 - coord_feat.shape[1]), jnp.float32)],
        axis=1)

    We1a = We1[:d]
    We1b = We1[d:2 * d]
    wr = We1[2 * d:2 * d + 1]
    We1e = We1[2 * d + 1:]

    t1, t2 = _build_tables(node_feat, coordp, We1a, be1.reshape(1, h), We1b,
                           bn=2000)
    gx = _gather_edges(t1, t2, src2d, dst2d, e_pad, d)
    mh, mx = _edge_mlp(gx, edge_feat, wr, We1e, We2, be2.reshape(1, h),
                       Wc1, bc1.reshape(1, h), Wc2.reshape(1, h), be=2000, h=h)
    zh = jnp.zeros((n, h), jnp.float32)
    zx = jnp.zeros((n, 16), jnp.float32)
    oh, ox = _scatter_edges(mh, mx, dst2d, zh, zx, n, e, e_pad, h)
    hx, co = _node_mlp(node_feat, coordp, oh[0], oh[1], ox[0], ox[1],
                       Wn1[:d], Wn1[d:], bn1.reshape(1, h), Wn2,
                       bn2.reshape(1, o), bn=2000, h=h, o=o)
    return hx, co[:, :coord_feat.shape[1]]


# speedup vs baseline: 3.4964x; 1.0006x over previous
import functools

import jax
import jax.numpy as jnp
from jax import lax
from jax.experimental import pallas as pl
from jax.experimental.pallas import tpu as pltpu
from jax.experimental.pallas import tpu_sc as plsc

NW = 32
CHUNK = 128


def _cdiv(a, b):
    return (a + b - 1) // b


def _build_tables(node_feat, coordp, We1a, be1, We1b, bn):
    n = node_feat.shape[0]
    d = node_feat.shape[1]

    def body(nf_ref, cp_ref, wa_ref, ba_ref, wb_ref, t1_ref, t2_ref):
        nf = nf_ref[...]
        t1_ref[:, :d] = jnp.dot(nf, wa_ref[...],
                                preferred_element_type=jnp.float32) + ba_ref[...]
        t1_ref[:, d:] = cp_ref[...]
        t2_ref[:, :d] = jnp.dot(nf, wb_ref[...],
                                preferred_element_type=jnp.float32)
        t2_ref[:, d:] = cp_ref[...]

    grid = (n // bn,)
    out = pl.pallas_call(
        body,
        grid=grid,
        in_specs=[
            pl.BlockSpec((bn, d), lambda i: (i, 0)),
            pl.BlockSpec((bn, 16), lambda i: (i, 0)),
            pl.BlockSpec((d, d), lambda i: (0, 0)),
            pl.BlockSpec((1, d), lambda i: (0, 0)),
            pl.BlockSpec((d, d), lambda i: (0, 0)),
        ],
        out_specs=[
            pl.BlockSpec((bn, d + 16), lambda i: (i, 0)),
            pl.BlockSpec((bn, d + 16), lambda i: (i, 0)),
        ],
        out_shape=[
            jax.ShapeDtypeStruct((n, d + 16), jnp.float32),
            jax.ShapeDtypeStruct((n, d + 16), jnp.float32),
        ],
    )(node_feat, coordp, We1a, be1, We1b)
    return out


def _gather_edges(t1, t2, src2d, dst2d, e_pad, d):
    nchunk = e_pad // CHUNK
    per_w = nchunk // NW
    w16 = d + 16
    mesh = plsc.VectorSubcoreMesh(core_axis_name="c", subcore_axis_name="s")

    @functools.partial(
        pl.kernel,
        out_type=jax.ShapeDtypeStruct((e_pad, w16), jnp.float32),
        mesh=mesh,
        scratch_types=[
            pltpu.VMEM((per_w, CHUNK), jnp.int32),
            pltpu.VMEM((per_w, CHUNK), jnp.int32),
            pltpu.VMEM((2, CHUNK, w16), jnp.float32),
            pltpu.VMEM((2, CHUNK, w16), jnp.float32),
            pltpu.SemaphoreType.DMA,
            pltpu.SemaphoreType.DMA,
            pltpu.SemaphoreType.DMA,
            pltpu.SemaphoreType.DMA,
        ],
        compiler_params=pltpu.CompilerParams(use_tc_tiling_on_sc=False, skip_device_barrier=True),
    )
    def gath(t1_hbm, t2_hbm, src_hbm, dst_hbm, gx_hbm,
             sidx, didx, r1, r2, s1a, s1b, s2a, s2b):
        w = lax.axis_index("s") * 2 + lax.axis_index("c")
        cbase = w * per_w
        pltpu.sync_copy(src_hbm.at[pl.ds(cbase, per_w)], sidx)
        pltpu.sync_copy(dst_hbm.at[pl.ds(cbase, per_w)], didx)
        sems1 = (s1a, s1b)
        sems2 = (s2a, s2b)

        def issue(c, b):
            pltpu.async_copy(t1_hbm.at[sidx.at[c]], r1.at[b], sems1[b])
            pltpu.async_copy(t2_hbm.at[didx.at[c]], r2.at[b], sems2[b])

        def wait(b):
            pltpu.make_async_copy(t1_hbm.at[sidx.at[0]], r1.at[b],
                                  sems1[b]).wait()
            pltpu.make_async_copy(t2_hbm.at[didx.at[0]], r2.at[b],
                                  sems2[b]).wait()

        issue(0, 0)
        issue(1, 1)

        @pl.loop(0, _cdiv(per_w, 2))
        def _(i):
            for b in range(2):
                c = 2 * i + b

                @pl.when(c < per_w)
                def _():
                    wait(b)

                    @pl.loop(0, CHUNK)
                    def _(r):
                        row = (b, pl.ds(r, 1))
                        for j in range(d // 16):
                            sl = row + (pl.ds(16 * j, 16),)
                            r1[sl] = r1[sl] + r2[sl]
                        xsl = row + (pl.ds(d, 16),)
                        r1[xsl] = r1[xsl] - r2[xsl]

                    pltpu.sync_copy(
                        r1.at[b], gx_hbm.at[pl.ds((cbase + c) * CHUNK, CHUNK)])

                    @pl.when(c + 2 < per_w)
                    def _():
                        issue(c + 2, b)

    return gath(t1, t2, src2d, dst2d)


def _edge_mlp(gx, edge_feat, wr, We1e, We2, be2, Wc1, bc1, wc2r, be, h):
    e = edge_feat.shape[0]
    ed = edge_feat.shape[1]

    def body(gx_ref, ef_ref, wr_ref, w1e_ref, w2_ref, b2_ref,
             wc1_ref, bc1_ref, wc2_ref, mh_ref, mx_ref):
        g = gx_ref[:, :h]
        xdv = gx_ref[:, h:]
        radial = jnp.sum(xdv * xdv, axis=1, keepdims=True)
        pre = g + radial * wr_ref[...] + jnp.dot(
            ef_ref[...], w1e_ref[...], preferred_element_type=jnp.float32)
        z = pre * jax.nn.sigmoid(pre)
        m = jnp.dot(z, w2_ref[...], preferred_element_type=jnp.float32) + b2_ref[...]
        m = m * jax.nn.sigmoid(m)
        t = jnp.dot(m, wc1_ref[...], preferred_element_type=jnp.float32) + bc1_ref[...]
        t = t * jax.nn.sigmoid(t)
        coef = jnp.sum(t * wc2_ref[...], axis=1, keepdims=True)
        rnorm = coef / (jnp.sqrt(radial) + 1e-30)
        mh_ref[...] = m
        lane = lax.broadcasted_iota(jnp.int32, (1, 16), 1)
        mx_ref[...] = rnorm * xdv + jnp.where(lane == 3, 1.0, 0.0)

    grid = (e // be,)
    return pl.pallas_call(
        body,
        grid=grid,
        in_specs=[
            pl.BlockSpec((be, h + 16), lambda i: (i, 0)),
            pl.BlockSpec((be, ed), lambda i: (i, 0)),
            pl.BlockSpec((1, h), lambda i: (0, 0)),
            pl.BlockSpec((ed, h), lambda i: (0, 0)),
            pl.BlockSpec((h, h), lambda i: (0, 0)),
            pl.BlockSpec((1, h), lambda i: (0, 0)),
            pl.BlockSpec((h, h), lambda i: (0, 0)),
            pl.BlockSpec((1, h), lambda i: (0, 0)),
            pl.BlockSpec((1, h), lambda i: (0, 0)),
        ],
        out_specs=[
            pl.BlockSpec((be, h), lambda i: (i, 0)),
            pl.BlockSpec((be, 16), lambda i: (i, 0)),
        ],
        out_shape=[
            jax.ShapeDtypeStruct((e, h), jnp.float32),
            jax.ShapeDtypeStruct((e, 16), jnp.float32),
        ],
    )(gx, edge_feat, wr, We1e, We2, be2, Wc1, bc1, wc2r)


def _scatter_edges(mh, mx, dst2d, zh, zx, n, e, e_pad, h):
    nchunk = e // CHUNK
    nchunk_pad = e_pad // CHUNK
    per_w = nchunk_pad // NW
    rows_per_sub = n // 16
    mesh = plsc.VectorSubcoreMesh(core_axis_name="c", subcore_axis_name="s")

    @functools.partial(
        pl.kernel,
        out_type=(
            jax.ShapeDtypeStruct((2, n, h), jnp.float32),
            jax.ShapeDtypeStruct((2, n, 16), jnp.float32),
        ),
        mesh=mesh,
        scratch_types=[
            pltpu.VMEM((2, CHUNK), jnp.int32),
            pltpu.VMEM((2, CHUNK, h), jnp.float32),
            pltpu.VMEM((2, CHUNK, 16), jnp.float32),
            pltpu.VMEM_SHARED((n, h), jnp.float32),
            pltpu.VMEM_SHARED((n, 16), jnp.float32),
            pltpu.SemaphoreType.DMA,
            pltpu.SemaphoreType.DMA,
            pltpu.SemaphoreType.DMA,
            pltpu.SemaphoreType.DMA,
            pltpu.SemaphoreType.DMA,
            pltpu.SemaphoreType.DMA,
        ],
        compiler_params=pltpu.CompilerParams(use_tc_tiling_on_sc=False, skip_device_barrier=True),
    )
    def scat(mh_hbm, mx_hbm, dst_hbm, zh_hbm, zx_hbm, oh_hbm, ox_hbm,
             dv, bh, bx, acch, accx, sha, shb, sxa, sxb, sdva, sdvb):
        c_id = lax.axis_index("c")
        s_id = lax.axis_index("s")
        w = s_id * 2 + c_id
        cbase = w * per_w
        rb = s_id * rows_per_sub
        pltpu.sync_copy(zh_hbm.at[pl.ds(rb, rows_per_sub)],
                        acch.at[pl.ds(rb, rows_per_sub)])
        pltpu.sync_copy(zx_hbm.at[pl.ds(rb, rows_per_sub)],
                        accx.at[pl.ds(rb, rows_per_sub)])
        plsc.subcore_barrier()

        semsh = (sha, shb)
        semsx = (sxa, sxb)
        semsd = (sdva, sdvb)

        @pl.loop(0, per_w)
        def _(i):
            @pl.when(cbase + i < nchunk)
            def _():
                base = (cbase + i) * CHUNK
                pltpu.sync_copy(dst_hbm.at[cbase + i], dv.at[0])
                pltpu.sync_copy(mh_hbm.at[pl.ds(base, CHUNK)], bh.at[0])
                pltpu.sync_copy(mx_hbm.at[pl.ds(base, CHUNK)], bx.at[0])
                pltpu.sync_copy(bh.at[0], acch.at[dv.at[0]], add=True)
                pltpu.sync_copy(bx.at[0], accx.at[dv.at[0]], add=True)

        plsc.subcore_barrier()
        pltpu.sync_copy(acch.at[pl.ds(rb, rows_per_sub)],
                        oh_hbm.at[c_id, pl.ds(rb, rows_per_sub)])
        pltpu.sync_copy(accx.at[pl.ds(rb, rows_per_sub)],
                        ox_hbm.at[c_id, pl.ds(rb, rows_per_sub)])

    return scat(mh, mx, dst2d, zh, zx)


def _node_mlp(node_feat, coordp, h0, h1, x0, x1, Wn1a, Wn1b, bn1, Wn2, bn2,
              bn, h, o):
    n = node_feat.shape[0]
    d = node_feat.shape[1]

    def body(nf_ref, cp_ref, h0_ref, h1_ref, x0_ref, x1_ref,
             wa_ref, wb_ref, b1_ref, w2_ref, b2_ref, hx_ref, co_ref):
        hn = h0_ref[...] + h1_ref[...]
        sx = x0_ref[...] + x1_ref[...]
        cnt = sx[:, 3:4]
        co_ref[...] = cp_ref[...] + sx / jnp.maximum(cnt, 1.0)
        z = (jnp.dot(nf_ref[...], wa_ref[...], preferred_element_type=jnp.float32)
             + jnp.dot(hn, wb_ref[...], preferred_element_type=jnp.float32)
             + b1_ref[...])
        z = z * jax.nn.sigmoid(z)
        hx_ref[...] = jnp.dot(z, w2_ref[...],
                              preferred_element_type=jnp.float32) + b2_ref[...]

    grid = (n // bn,)
    return pl.pallas_call(
        body,
        grid=grid,
        in_specs=[
            pl.BlockSpec((bn, d), lambda i: (i, 0)),
            pl.BlockSpec((bn, 16), lambda i: (i, 0)),
            pl.BlockSpec((bn, h), lambda i: (i, 0)),
            pl.BlockSpec((bn, h), lambda i: (i, 0)),
            pl.BlockSpec((bn, 16), lambda i: (i, 0)),
            pl.BlockSpec((bn, 16), lambda i: (i, 0)),
            pl.BlockSpec((d, h), lambda i: (0, 0)),
            pl.BlockSpec((h, h), lambda i: (0, 0)),
            pl.BlockSpec((1, h), lambda i: (0, 0)),
            pl.BlockSpec((h, o), lambda i: (0, 0)),
            pl.BlockSpec((1, o), lambda i: (0, 0)),
        ],
        out_specs=[
            pl.BlockSpec((bn, o), lambda i: (i, 0)),
            pl.BlockSpec((bn, 16), lambda i: (i, 0)),
        ],
        out_shape=[
            jax.ShapeDtypeStruct((n, o), jnp.float32),
            jax.ShapeDtypeStruct((n, 16), jnp.float32),
        ],
    )(node_feat, coordp, h0, h1, x0, x1, Wn1a, Wn1b, bn1, Wn2, bn2)


def kernel(node_feat, coord_feat, edge_index, edge_feat,
           We1, be1, We2, be2, Wc1, bc1, Wc2, Wn1, bn1, Wn2, bn2):
    n, d = node_feat.shape
    e = edge_index.shape[1]
    h = We2.shape[0]
    o = Wn2.shape[1]

    nchunk = _cdiv(e, CHUNK)
    nchunk_pad = _cdiv(nchunk, NW) * NW
    e_pad = nchunk_pad * CHUNK

    pad = jnp.zeros((2, e_pad - e), jnp.int32)
    eidx = jnp.concatenate([edge_index, pad], axis=1)
    src2d = eidx[0].reshape(nchunk_pad, CHUNK)
    dst2d = eidx[1].reshape(nchunk_pad, CHUNK)

    coordp = jnp.concatenate(
        [coord_feat, jnp.zeros((n, 16 - coord_feat.shape[1]), jnp.float32)],
        axis=1)

    We1a = We1[:d]
    We1b = We1[d:2 * d]
    wr = We1[2 * d:2 * d + 1]
    We1e = We1[2 * d + 1:]

    t1, t2 = _build_tables(node_feat, coordp, We1a, be1.reshape(1, h), We1b,
                           bn=2000)
    gx = _gather_edges(t1, t2, src2d, dst2d, e_pad, d)
    mh, mx = _edge_mlp(gx, edge_feat, wr, We1e, We2, be2.reshape(1, h),
                       Wc1, bc1.reshape(1, h), Wc2.reshape(1, h), be=2000, h=h)
    zh = jnp.zeros((n, h), jnp.float32)
    zx = jnp.zeros((n, 16), jnp.float32)
    oh, ox = _scatter_edges(mh, mx, dst2d, zh, zx, n, e, e_pad, h)
    hx, co = _node_mlp(node_feat, coordp, oh[0], oh[1], ox[0], ox[1],
                       Wn1[:d], Wn1[d:], bn1.reshape(1, h), Wn2,
                       bn2.reshape(1, o), bn=2000, h=h, o=o)
    return hx, co[:, :coord_feat.shape[1]]
